# baseline (speedup 1.0000x reference)
import functools

import jax
import jax.numpy as jnp
from jax import lax
from jax.experimental import pallas as pl
from jax.experimental.pallas import tpu as pltpu
from jax.experimental.pallas import tpu_sc as plsc

VOCAB = 100000
HIDDEN = 64
B = 16
S = 50
EOS = 2
NUM_IDS = B * S


_NC = 2
_NS = 16
_ROWS_PER_W = 32
_ACTIVE = NUM_IDS // _ROWS_PER_W


def _sc_gather_body(idx_hbm, table_hbm, out_hbm, idx_v, rows_v, sem):
    wid = lax.axis_index("s") * _NC + lax.axis_index("c")

    @pl.when(wid < _ACTIVE)
    def _():
        base = wid * _ROWS_PER_W
        pltpu.sync_copy(idx_hbm.at[pl.ds(base, _ROWS_PER_W)], idx_v)
        pltpu.async_copy(table_hbm.at[idx_v], rows_v, sem).wait()
        for r in range(_ROWS_PER_W):
            for c in range(HIDDEN // 16):
                sl = pl.ds(c * 16, 16)
                rows_v[r, sl] = rows_v[r, sl] + 0.5
        pltpu.sync_copy(rows_v, out_hbm.at[pl.ds(base, _ROWS_PER_W)])


@functools.lru_cache(maxsize=1)
def _sc_gather():
    return functools.partial(
        pl.kernel,
        mesh=plsc.VectorSubcoreMesh(core_axis_name="c", subcore_axis_name="s"),
        out_type=jax.ShapeDtypeStruct((NUM_IDS, HIDDEN), jnp.float32),
        compiler_params=pltpu.CompilerParams(use_tc_tiling_on_sc=False),
        scratch_types=[
            pltpu.VMEM((_ROWS_PER_W,), jnp.int32),
            pltpu.VMEM((_ROWS_PER_W, HIDDEN), jnp.float32),
            pltpu.SemaphoreType.DMA,
        ],
    )(_sc_gather_body)


_NSEM = 8


def _fill_body(out_ref, slab0, slab1, sems):
    for slab in (slab0, slab1):
        slab[...] = jnp.full((S, VOCAB), -1000.0, jnp.float32)
        col = lax.broadcasted_iota(jnp.int32, (1, 128), 1)
        slab[S - 1:S, 0:128] = jnp.where(col == EOS, 1000.0, -1000.0)
    slabs = (slab0, slab1)
    copies = [pltpu.make_async_copy(slabs[b % 2], out_ref.at[b],
                                    sems.at[b % _NSEM])
              for b in range(B)]
    for cp in copies:
        cp.start()
    for cp in copies:
        cp.wait()


_fill_logits = pl.pallas_call(
    _fill_body,
    out_specs=pl.BlockSpec(memory_space=pl.ANY),
    out_shape=jax.ShapeDtypeStruct((B, S, VOCAB), jnp.float32),
    scratch_shapes=[
        pltpu.VMEM((S, VOCAB), jnp.float32),
        pltpu.VMEM((S, VOCAB), jnp.float32),
        pltpu.SemaphoreType.DMA((_NSEM,)),
    ],
)


def kernel(input_ids, table):
    ids = input_ids.reshape(NUM_IDS)
    hidden = _sc_gather()(ids, table).reshape(B, S, HIDDEN)
    logits = _fill_logits()
    cache = jnp.zeros((1, 1, S, HIDDEN), jnp.float32)
    return (logits, hidden, cache, cache)


# speedup vs baseline: 2.2494x; 2.2494x over previous
import functools

import jax
import jax.numpy as jnp
from jax import lax
from jax.experimental import pallas as pl
from jax.experimental.pallas import tpu as pltpu
from jax.experimental.pallas import tpu_sc as plsc

VOCAB = 100000
HIDDEN = 64
B = 16
S = 50
EOS = 2
NUM_IDS = B * S


_NC = 2
_NS = 16
_ROWS_PER_W = 32
_ACTIVE = NUM_IDS // _ROWS_PER_W


def _sc_gather_body(idx_hbm, table_hbm, out_hbm, idx_v, rows_v, sem):
    wid = lax.axis_index("s") * _NC + lax.axis_index("c")

    @pl.when(wid < _ACTIVE)
    def _():
        base = wid * _ROWS_PER_W
        pltpu.sync_copy(idx_hbm.at[pl.ds(base, _ROWS_PER_W)], idx_v)
        pltpu.async_copy(table_hbm.at[idx_v], rows_v, sem).wait()
        for r in range(_ROWS_PER_W):
            for c in range(HIDDEN // 16):
                sl = pl.ds(c * 16, 16)
                rows_v[r, sl] = rows_v[r, sl] + 0.5
        pltpu.sync_copy(rows_v, out_hbm.at[pl.ds(base, _ROWS_PER_W)])


@functools.lru_cache(maxsize=1)
def _sc_gather():
    return functools.partial(
        pl.kernel,
        mesh=plsc.VectorSubcoreMesh(core_axis_name="c", subcore_axis_name="s"),
        out_type=jax.ShapeDtypeStruct((NUM_IDS, HIDDEN), jnp.float32),
        compiler_params=pltpu.CompilerParams(use_tc_tiling_on_sc=False),
        scratch_types=[
            pltpu.VMEM((_ROWS_PER_W,), jnp.int32),
            pltpu.VMEM((_ROWS_PER_W, HIDDEN), jnp.float32),
            pltpu.SemaphoreType.DMA,
        ],
    )(_sc_gather_body)


def _fill_body(out_ref):
    out_ref[...] = jnp.full((1, B, VOCAB), -1000.0, jnp.float32)

    @pl.when(pl.program_id(0) == S - 1)
    def _():
        col = lax.broadcasted_iota(jnp.int32, (1, B, 128), 2)
        out_ref[0:1, :, 0:128] = jnp.where(col == EOS, 1000.0, -1000.0)


_fill_logits = pl.pallas_call(
    _fill_body,
    grid=(S,),
    out_specs=pl.BlockSpec((1, B, VOCAB), lambda s: (s, 0, 0)),
    out_shape=jax.ShapeDtypeStruct((S, B, VOCAB), jnp.float32),
    compiler_params=pltpu.CompilerParams(
        dimension_semantics=("arbitrary",)),
)


def kernel(input_ids, table):
    ids_t = input_ids.T.reshape(NUM_IDS)
    hidden_t = _sc_gather()(ids_t, table).reshape(S, B, HIDDEN)
    hidden = hidden_t.transpose(1, 0, 2)
    logits = _fill_logits().transpose(1, 0, 2)
    cache = jnp.zeros((1, 1, S, HIDDEN), jnp.float32)
    return (logits, hidden, cache, cache)


# speedup vs baseline: 2.2537x; 1.0019x over previous
import functools

import jax
import jax.numpy as jnp
from jax import lax
from jax.experimental import pallas as pl
from jax.experimental.pallas import tpu as pltpu
from jax.experimental.pallas import tpu_sc as plsc

VOCAB = 100000
HIDDEN = 64
B = 16
S = 50
EOS = 2
NUM_IDS = B * S


_NC = 2
_NS = 16
_ROWS_PER_W = 32
_ACTIVE = NUM_IDS // _ROWS_PER_W


def _sc_gather_body(idx_hbm, table_hbm, out_hbm, idx_v, rows_v, sem):
    wid = lax.axis_index("s") * _NC + lax.axis_index("c")

    @pl.when(wid < _ACTIVE)
    def _():
        base = wid * _ROWS_PER_W
        pltpu.sync_copy(idx_hbm.at[pl.ds(base, _ROWS_PER_W)], idx_v)
        pltpu.async_copy(table_hbm.at[idx_v], rows_v, sem).wait()
        for r in range(_ROWS_PER_W):
            for c in range(HIDDEN // 16):
                sl = pl.ds(c * 16, 16)
                rows_v[r, sl] = rows_v[r, sl] + 0.5
        pltpu.sync_copy(rows_v, out_hbm.at[pl.ds(base, _ROWS_PER_W)])


@functools.lru_cache(maxsize=1)
def _sc_gather():
    return functools.partial(
        pl.kernel,
        mesh=plsc.VectorSubcoreMesh(core_axis_name="c", subcore_axis_name="s"),
        out_type=jax.ShapeDtypeStruct((NUM_IDS, HIDDEN), jnp.float32),
        compiler_params=pltpu.CompilerParams(use_tc_tiling_on_sc=False),
        scratch_types=[
            pltpu.VMEM((_ROWS_PER_W,), jnp.int32),
            pltpu.VMEM((_ROWS_PER_W, HIDDEN), jnp.float32),
            pltpu.SemaphoreType.DMA,
        ],
    )(_sc_gather_body)


def _fill_body(out_ref):
    out_ref[...] = jnp.full((1, B, VOCAB), -1000.0, jnp.float32)

    @pl.when(pl.program_id(0) == S - 1)
    def _():
        col = lax.broadcasted_iota(jnp.int32, (1, B, 128), 2)
        out_ref[0:1, :, 0:128] = jnp.where(col == EOS, 1000.0, -1000.0)


_fill_logits = pl.pallas_call(
    _fill_body,
    grid=(S,),
    out_specs=pl.BlockSpec((1, B, VOCAB), lambda s: (s, 0, 0)),
    out_shape=jax.ShapeDtypeStruct((S, B, VOCAB), jnp.float32),
    compiler_params=pltpu.CompilerParams(
        dimension_semantics=("arbitrary",)),
)


def kernel(input_ids, table):
    ids = input_ids.reshape(NUM_IDS)
    hidden = _sc_gather()(ids, table).reshape(B, S, HIDDEN)
    logits = _fill_logits().transpose(1, 0, 2)
    cache = jnp.zeros((1, 1, S, HIDDEN), jnp.float32)
    return (logits, hidden, cache, cache)
